# Initial kernel scaffold; baseline (speedup 1.0000x reference)
import jax
import jax.numpy as jnp
from jax.experimental import pallas as pl


def kernel(output, target, anchors):
    raise NotImplementedError("write your pallas kernel here")


# speedup vs baseline: 28.8312x; 28.8312x over previous
import functools

import jax
import jax.numpy as jnp
from jax import lax
from jax.experimental import pallas as pl
from jax.experimental.pallas import tpu as pltpu


def _yolo_body(o_ref, t_ref, a_ref, out_ref, *, Bn, An, Cc, Hh, Ww, T):
    b = pl.program_id(0)
    HW = Hh * Ww
    f32 = jnp.float32

    tt = t_ref[0]
    b1 = tt[:, 1:2]
    cx = tt[:, 0:1] * Ww
    cy = b1 * Hh
    w = tt[:, 2:3] * Ww
    h = tt[:, 3:4] * Hh
    tcl = tt[:, 4:5].astype(jnp.int32).astype(f32)

    r_i = lax.broadcasted_iota(jnp.int32, (T, T), 0)
    c_i = lax.broadcasted_iota(jnp.int32, (T, T), 1)
    iszero = jnp.where(b1 == 0.0, 1.0, 0.0)
    zcnt = jnp.dot(jnp.where(c_i <= r_i, 1.0, 0.0).astype(f32), iszero,
                   preferred_element_type=f32)
    valid_b = zcnt == 0.0
    valid_f = jnp.where(valid_b, 1.0, 0.0)

    aw = [a_ref[i, 0] for i in range(An)]
    ah = [a_ref[i, 1] for i in range(An)]

    def anchor_iou(i):
        inter = jnp.minimum(w, aw[i]) * jnp.minimum(h, ah[i])
        return inter / (w * h + aw[i] * ah[i] - inter)

    i0, i1, i2 = anchor_iou(0), anchor_iou(1), anchor_iou(2)
    best = jnp.where(i1 > i0, 1, 0)
    best = jnp.where(i2 > jnp.maximum(i0, i1), 2, best)

    cif = jnp.floor(cx)
    cjf = jnp.floor(cy)
    cjci = cjf.astype(jnp.int32) * Ww + cif.astype(jnp.int32)
    aw_best = jnp.where(best == 0, aw[0], jnp.where(best == 1, aw[1], aw[2]))
    ah_best = jnp.where(best == 0, ah[0], jnp.where(best == 1, ah[1], ah[2]))
    tb0 = cx - cif
    tb1 = cy - cjf
    tb2 = jnp.log(w / aw_best)
    tb3 = jnp.log(h / ah_best)

    U = jnp.where(c_i > r_i, 1.0, 0.0).astype(f32)

    hw_iota = lax.broadcasted_iota(jnp.int32, (1, HW), 1)
    gx = (hw_iota % Ww).astype(f32)
    gy = (hw_iota // Ww).astype(f32)

    Sb = w * h
    box_l = cx - w * 0.5
    box_r = cx + w * 0.5
    box_t = cy - h * 0.5
    box_b = cy + h * 0.5

    loss_box = jnp.float32(0.0)
    loss_conf = jnp.float32(0.0)
    loss_cls = jnp.float32(0.0)

    for a in range(An):
        base = a * (Cc + 5)
        tx = o_ref[0, base + 0:base + 1, :]
        ty = o_ref[0, base + 1:base + 2, :]
        tw = o_ref[0, base + 2:base + 3, :]
        th = o_ref[0, base + 3:base + 4, :]
        tcf = o_ref[0, base + 4:base + 5, :]
        sx = jax.nn.sigmoid(tx)
        sy = jax.nn.sigmoid(ty)
        ew = jnp.exp(tw)
        eh = jnp.exp(th)
        pc = jax.nn.sigmoid(tcf)

        qa = (An * b + a) // Bn
        awq = jnp.where(qa == 0, aw[0], jnp.where(qa == 1, aw[1], aw[2]))
        ahq = jnp.where(qa == 0, ah[0], jnp.where(qa == 1, ah[1], ah[2]))

        px = sx + gx
        py = sy + gy
        pw = ew * awq
        ph = eh * ahq
        Sa = pw * ph
        pl_ = px - pw * 0.5
        pr_ = px + pw * 0.5
        pt_ = py - ph * 0.5
        pb_ = py + ph * 0.5

        x1 = jnp.maximum(pl_, box_l)
        x2 = jnp.minimum(pr_, box_r)
        y1 = jnp.maximum(pt_, box_t)
        y2 = jnp.minimum(pb_, box_b)
        inter = jnp.maximum(x2 - x1, 0.0) * jnp.maximum(y2 - y1, 0.0)
        ig_pred = (3.0 * inter > Sa + Sb) & valid_b
        ign = jnp.max(jnp.where(ig_pred, 1.0, 0.0), axis=0, keepdims=True)

        sel = (cjci == hw_iota) & (best == a) & valid_b
        sel_f = jnp.where(sel, 1.0, 0.0).astype(f32)
        suffix = jnp.dot(U, sel_f, preferred_element_type=f32)
        live = sel_f * jnp.where(suffix == 0.0, 1.0, 0.0)
        assigned = jnp.sum(live, axis=0, keepdims=True)
        tb0d = jnp.sum(live * tb0, axis=0, keepdims=True)
        tb1d = jnp.sum(live * tb1, axis=0, keepdims=True)
        tb2d = jnp.sum(live * tb2, axis=0, keepdims=True)
        tb3d = jnp.sum(live * tb3, axis=0, keepdims=True)
        tcld = jnp.sum(live * tcl, axis=0, keepdims=True)

        for pk, tkd in ((sx, tb0d), (sy, tb1d), (ew, tb2d), (eh, tb3d)):
            d = pk * assigned - tkd
            loss_box += jnp.sum(d * d)

        notign = 1.0 - ign
        nota = 1.0 - assigned
        dconf = pc - 1.0
        loss_conf += jnp.sum(pc * pc * notign * nota) + jnp.sum(
            assigned * dconf * dconf)

        cls = o_ref[0, base + 5:base + 5 + Cc, :]
        mx = jnp.max(cls, axis=0, keepdims=True)
        ssum = jnp.sum(jnp.exp(cls - mx), axis=0, keepdims=True)
        lse = mx + jnp.log(ssum)
        c_iota = lax.broadcasted_iota(jnp.int32, (Cc, HW), 0).astype(f32)
        picked = jnp.sum(cls * jnp.where(c_iota == tcld, 1.0, 0.0),
                         axis=0, keepdims=True)
        loss_cls += jnp.sum(assigned * (picked - lse))

    total = loss_box * 0.5 + loss_conf - loss_cls

    @pl.when(b == 0)
    def _():
        out_ref[:, :] = jnp.zeros((1, 1), jnp.float32)

    out_ref[:, :] = out_ref[:, :] + total


def kernel(output, target, anchors):
    Bn, ch, Hh, Ww = output.shape
    An = anchors.shape[0]
    Cc = ch // An - 5
    T = target.shape[1] // 5
    HW = Hh * Ww

    o3 = output.reshape(Bn, ch, HW)
    t3 = target.reshape(Bn, T, 5)

    body = functools.partial(_yolo_body, Bn=Bn, An=An, Cc=Cc, Hh=Hh, Ww=Ww,
                             T=T)
    res = pl.pallas_call(
        body,
        grid=(Bn,),
        in_specs=[
            pl.BlockSpec((1, ch, HW), lambda b: (b, 0, 0)),
            pl.BlockSpec((1, T, 5), lambda b: (b, 0, 0)),
            pl.BlockSpec((An, 2), lambda b: (0, 0)),
        ],
        out_specs=pl.BlockSpec((1, 1), lambda b: (0, 0)),
        out_shape=jax.ShapeDtypeStruct((1, 1), jnp.float32),
        compiler_params=pltpu.CompilerParams(
            dimension_semantics=("arbitrary",)),
    )(o3, t3, anchors)
    return res[0, 0]


# speedup vs baseline: 33.3466x; 1.1566x over previous
import functools

import jax
import jax.numpy as jnp
from jax import lax
from jax.experimental import pallas as pl
from jax.experimental.pallas import tpu as pltpu


def _yolo_body(o_ref, t_ref, a_ref, out_ref, *, Bn, An, Cc, Hh, Ww, T):
    b = pl.program_id(0)
    HW = Hh * Ww
    f32 = jnp.float32

    tt = t_ref[0]
    b1 = tt[:, 1:2]
    cx = tt[:, 0:1] * Ww
    cy = b1 * Hh
    w = tt[:, 2:3] * Ww
    h = tt[:, 3:4] * Hh
    tcl = tt[:, 4:5].astype(jnp.int32).astype(f32)

    r_i = lax.broadcasted_iota(jnp.int32, (T, T), 0)
    c_i = lax.broadcasted_iota(jnp.int32, (T, T), 1)
    iszero = jnp.where(b1 == 0.0, 1.0, 0.0)
    zcnt = jnp.dot(jnp.where(c_i <= r_i, 1.0, 0.0).astype(f32), iszero,
                   preferred_element_type=f32)
    valid_b = zcnt == 0.0
    valid_f = jnp.where(valid_b, 1.0, 0.0)

    aw = [a_ref[i, 0] for i in range(An)]
    ah = [a_ref[i, 1] for i in range(An)]

    def anchor_iou(i):
        inter = jnp.minimum(w, aw[i]) * jnp.minimum(h, ah[i])
        return inter / (w * h + aw[i] * ah[i] - inter)

    i0, i1, i2 = anchor_iou(0), anchor_iou(1), anchor_iou(2)
    best = jnp.where(i1 > i0, 1, 0)
    best = jnp.where(i2 > jnp.maximum(i0, i1), 2, best)

    cif = jnp.floor(cx)
    cjf = jnp.floor(cy)
    cjci = cjf.astype(jnp.int32) * Ww + cif.astype(jnp.int32)
    aw_best = jnp.where(best == 0, aw[0], jnp.where(best == 1, aw[1], aw[2]))
    ah_best = jnp.where(best == 0, ah[0], jnp.where(best == 1, ah[1], ah[2]))
    tb0 = cx - cif
    tb1 = cy - cjf
    tb2 = jnp.log(w / aw_best)
    tb3 = jnp.log(h / ah_best)

    U = jnp.where(c_i > r_i, 1.0, 0.0).astype(f32)

    hw_iota = lax.broadcasted_iota(jnp.int32, (1, HW), 1)
    gx = (hw_iota % Ww).astype(f32)
    gy = (hw_iota // Ww).astype(f32)

    Sb = w * h
    box_l = cx - w * 0.5
    box_r = cx + w * 0.5
    box_t = cy - h * 0.5
    box_b = cy + h * 0.5

    loss_box = jnp.float32(0.0)
    loss_conf = jnp.float32(0.0)
    loss_cls = jnp.float32(0.0)

    for a in range(An):
        base = a * (Cc + 5)
        tx = o_ref[0, base + 0:base + 1, :]
        ty = o_ref[0, base + 1:base + 2, :]
        tw = o_ref[0, base + 2:base + 3, :]
        th = o_ref[0, base + 3:base + 4, :]
        tcf = o_ref[0, base + 4:base + 5, :]
        sx = jax.nn.sigmoid(tx)
        sy = jax.nn.sigmoid(ty)
        ew = jnp.exp(tw)
        eh = jnp.exp(th)
        pc = jax.nn.sigmoid(tcf)

        qa = (An * b + a) // Bn
        awq = jnp.where(qa == 0, aw[0], jnp.where(qa == 1, aw[1], aw[2]))
        ahq = jnp.where(qa == 0, ah[0], jnp.where(qa == 1, ah[1], ah[2]))

        px = sx + gx
        py = sy + gy
        pw = ew * awq
        ph = eh * ahq
        Sa = pw * ph
        pl_ = px - pw * 0.5
        pr_ = px + pw * 0.5
        pt_ = py - ph * 0.5
        pb_ = py + ph * 0.5

        x1 = jnp.maximum(pl_, box_l)
        x2 = jnp.minimum(pr_, box_r)
        y1 = jnp.maximum(pt_, box_t)
        y2 = jnp.minimum(pb_, box_b)
        inter = jnp.maximum(x2 - x1, 0.0) * jnp.maximum(y2 - y1, 0.0)
        ig_pred = (3.0 * inter > Sa + Sb) & valid_b
        ign = jnp.max(jnp.where(ig_pred, 1.0, 0.0), axis=0, keepdims=True)

        sel = (cjci == hw_iota) & (best == a) & valid_b
        sel_f = jnp.where(sel, 1.0, 0.0).astype(f32)
        suffix = jnp.dot(U, sel_f, preferred_element_type=f32)
        live = sel_f * jnp.where(suffix == 0.0, 1.0, 0.0)
        assigned = jnp.sum(live, axis=0, keepdims=True)
        islive = jnp.sum(live, axis=1, keepdims=True)

        p5 = jnp.concatenate([sx, sy, ew, eh, pc], axis=0)
        dn = (((1,), (1,)), ((), ()))
        g = lax.dot_general(live, p5, dn, preferred_element_type=f32)
        for k, tbk in enumerate((tb0, tb1, tb2, tb3)):
            d = g[:, k:k + 1] - tbk
            loss_box += jnp.sum(islive * d * d)

        notign = 1.0 - ign
        nota = 1.0 - assigned
        dconf = g[:, 4:5] - 1.0
        loss_conf += jnp.sum(pc * pc * notign * nota) + jnp.sum(
            islive * dconf * dconf)

        cls = o_ref[0, base + 5:base + 5 + Cc, :]
        rows = lax.dot_general(live, cls, dn, preferred_element_type=f32)
        mx = jnp.max(rows, axis=1, keepdims=True)
        ssum = jnp.sum(jnp.exp(rows - mx), axis=1, keepdims=True)
        lse = mx + jnp.log(ssum)
        c_iota = lax.broadcasted_iota(jnp.int32, (T, Cc), 1).astype(f32)
        picked = jnp.sum(rows * jnp.where(c_iota == tcl, 1.0, 0.0),
                         axis=1, keepdims=True)
        loss_cls += jnp.sum(islive * (picked - lse))

    total = loss_box * 0.5 + loss_conf - loss_cls

    @pl.when(b == 0)
    def _():
        out_ref[:, :] = jnp.zeros((1, 1), jnp.float32)

    out_ref[:, :] = out_ref[:, :] + total


def kernel(output, target, anchors):
    Bn, ch, Hh, Ww = output.shape
    An = anchors.shape[0]
    Cc = ch // An - 5
    T = target.shape[1] // 5
    HW = Hh * Ww

    o3 = output.reshape(Bn, ch, HW)
    t3 = target.reshape(Bn, T, 5)

    body = functools.partial(_yolo_body, Bn=Bn, An=An, Cc=Cc, Hh=Hh, Ww=Ww,
                             T=T)
    res = pl.pallas_call(
        body,
        grid=(Bn,),
        in_specs=[
            pl.BlockSpec((1, ch, HW), lambda b: (b, 0, 0)),
            pl.BlockSpec((1, T, 5), lambda b: (b, 0, 0)),
            pl.BlockSpec((An, 2), lambda b: (0, 0)),
        ],
        out_specs=pl.BlockSpec((1, 1), lambda b: (0, 0)),
        out_shape=jax.ShapeDtypeStruct((1, 1), jnp.float32),
        compiler_params=pltpu.CompilerParams(
            dimension_semantics=("arbitrary",)),
    )(o3, t3, anchors)
    return res[0, 0]
